# Initial kernel scaffold; baseline (speedup 1.0000x reference)
import jax
import jax.numpy as jnp
from jax.experimental import pallas as pl


def kernel(x, edge_index, node_types, W_src, b_src, W_hid, b_hid, W_out, b_out, W_log, b_log):
    raise NotImplementedError("write your pallas kernel here")


# speedup vs baseline: 48.7467x; 48.7467x over previous
import functools

import jax
import jax.numpy as jnp
from jax import lax
from jax.experimental import pallas as pl
from jax.experimental.pallas import tpu as pltpu
from jax.experimental.pallas import tpu_sc as plsc

N = 100000
E = 3200000
D = 7
H = 16
OUT = 32
T = 7

NC = 2
NS = 16
NW = NC * NS
L = 128
K = 8
ROWS_PER_W = 800
CHUNKS = ROWS_PER_W // K
E_PAD = NW * ROWS_PER_W * L
BLK = 2048
N_PAD = 100096
GRID = -(-N_PAD // BLK)
TILE_ROWS = N_PAD // NS
ZROWS = TILE_ROWS // 8


def _typed_transform_body(x_ref, nt_ref, w_ref, o_ref):
    i = pl.program_id(0)
    xb = x_ref[...]
    tb = nt_ref[...]
    acc = jnp.zeros((BLK, H), jnp.float32)
    for t in range(T):
        yt = jnp.dot(xb, w_ref[t], preferred_element_type=jnp.float32)
        acc = acc + jnp.where(tb == t, yt, 0.0)
    row = i * BLK + lax.broadcasted_iota(jnp.int32, (BLK, 1), 0)
    o_ref[...] = jnp.where(row < N, acc, 0.0)


def _typed_transform(x, node_types, W_src):
    return pl.pallas_call(
        _typed_transform_body,
        grid=(GRID,),
        in_specs=[
            pl.BlockSpec((BLK, D), lambda i: (i, 0)),
            pl.BlockSpec((BLK, 1), lambda i: (i, 0)),
            pl.BlockSpec((T, D, H), lambda i: (0, 0, 0)),
        ],
        out_specs=pl.BlockSpec((BLK, H), lambda i: (i, 0)),
        out_shape=jax.ShapeDtypeStruct((N_PAD, H), jnp.float32),
    )(x, node_types.reshape(N, 1), W_src)


def _combine_body(p0_ref, p1_ref, b_ref, o_ref):
    o_ref[...] = jax.nn.leaky_relu(p0_ref[...] + p1_ref[...] + b_ref[...])


def _combine(p0, p1, b_src):
    return pl.pallas_call(
        _combine_body,
        grid=(GRID,),
        in_specs=[
            pl.BlockSpec((BLK, H), lambda i: (i, 0)),
            pl.BlockSpec((BLK, H), lambda i: (i, 0)),
            pl.BlockSpec((1, H), lambda i: (0, 0)),
        ],
        out_specs=pl.BlockSpec((BLK, H), lambda i: (i, 0)),
        out_shape=jax.ShapeDtypeStruct((N_PAD, H), jnp.float32),
    )(p0, p1, b_src.reshape(1, H))


def _final_body(p0_ref, p1_ref, wh_ref, bh_ref, wo_ref, bo_ref, wl_ref, bl_ref,
                o_ref, acc_ref):
    i = pl.program_id(0)

    @pl.when(i == 0)
    def _():
        acc_ref[...] = jnp.zeros_like(acc_ref)

    z = p0_ref[...] + p1_ref[...]
    hh = jax.nn.leaky_relu(
        jnp.dot(z, wh_ref[...], preferred_element_type=jnp.float32) + bh_ref[...])
    row = i * BLK + lax.broadcasted_iota(jnp.int32, (BLK, 1), 0)
    hh = jnp.where(row < N, hh, 0.0)
    acc_ref[...] += jnp.sum(hh, axis=0, keepdims=True)

    @pl.when(i == pl.num_programs(0) - 1)
    def _():
        m = acc_ref[...] / N
        g = jax.nn.sigmoid(
            jnp.dot(m, wo_ref[...], preferred_element_type=jnp.float32) + bo_ref[...])
        o_ref[...] = jax.nn.sigmoid(
            jnp.dot(g, wl_ref[...], preferred_element_type=jnp.float32) + bl_ref[...])


def _final(p0, p1, W_hid, b_hid, W_out, b_out, W_log, b_log):
    return pl.pallas_call(
        _final_body,
        grid=(GRID,),
        in_specs=[
            pl.BlockSpec((BLK, H), lambda i: (i, 0)),
            pl.BlockSpec((BLK, H), lambda i: (i, 0)),
            pl.BlockSpec((H, H), lambda i: (0, 0)),
            pl.BlockSpec((1, H), lambda i: (0, 0)),
            pl.BlockSpec((H, OUT), lambda i: (0, 0)),
            pl.BlockSpec((1, OUT), lambda i: (0, 0)),
            pl.BlockSpec((OUT, 1), lambda i: (0, 0)),
            pl.BlockSpec((1, 1), lambda i: (0, 0)),
        ],
        out_specs=pl.BlockSpec((1, 1), lambda i: (0, 0)),
        out_shape=jax.ShapeDtypeStruct((1, 1), jnp.float32),
        scratch_shapes=[pltpu.VMEM((1, H), jnp.float32)],
    )(p0, p1, W_hid, b_hid.reshape(1, H), W_out, b_out.reshape(1, OUT),
      W_log, b_log.reshape(1, 1))


def _sc_pass_kernel(src_hbm, dst_hbm, tab_hbm, out_hbm,
                    sidx, didx, rows, zbuf, acc, sem):
    c = lax.axis_index("c")
    s = lax.axis_index("s")
    wid = s * NC + c

    def zfill(i, carry):
        zbuf[i, :] = jnp.zeros((H,), jnp.float32)
        return carry
    lax.fori_loop(0, ZROWS, zfill, 0)
    tile_base = s * TILE_ROWS
    for q in range(TILE_ROWS // ZROWS):
        pltpu.sync_copy(zbuf, acc.at[pl.ds(tile_base + q * ZROWS, ZROWS), :])
    plsc.subcore_barrier()

    base_row = wid * ROWS_PER_W

    def chunk(g, carry):
        r0 = base_row + g * K
        pltpu.sync_copy(src_hbm.at[pl.ds(r0, K), :], sidx)
        pltpu.sync_copy(dst_hbm.at[pl.ds(r0, K), :], didx)
        cps = [pltpu.async_copy(tab_hbm.at[sidx.at[j]], rows.at[j], sem)
               for j in range(K)]
        for cp in cps:
            cp.wait()
        for j in range(K):
            pltpu.sync_copy(rows.at[j], acc.at[didx.at[j]], add=True)
        return carry
    lax.fori_loop(0, CHUNKS, chunk, 0)
    plsc.subcore_barrier()

    out_off = c * N_PAD + tile_base
    pltpu.sync_copy(acc.at[pl.ds(tile_base, TILE_ROWS), :],
                    out_hbm.at[pl.ds(out_off, TILE_ROWS), :])


def _sc_pass(src_rows, dst_rows, table):
    mesh = plsc.VectorSubcoreMesh(core_axis_name="c", subcore_axis_name="s")
    k = functools.partial(
        pl.kernel,
        out_type=jax.ShapeDtypeStruct((NC * N_PAD, H), jnp.float32),
        mesh=mesh,
        scratch_types=[
            pltpu.VMEM((K, L), jnp.int32),
            pltpu.VMEM((K, L), jnp.int32),
            pltpu.VMEM((K, L, H), jnp.float32),
            pltpu.VMEM((ZROWS, H), jnp.float32),
            pltpu.VMEM_SHARED((N_PAD, H), jnp.float32),
            pltpu.SemaphoreType.DMA,
        ],
        compiler_params=pltpu.CompilerParams(use_tc_tiling_on_sc=False),
    )(_sc_pass_kernel)
    parts = k(src_rows, dst_rows, table)
    return parts[:N_PAD], parts[N_PAD:]


def kernel(x, edge_index, node_types, W_src, b_src, W_hid, b_hid,
           W_out, b_out, W_log, b_log):
    src = edge_index[0]
    dst = edge_index[1]
    pad = jnp.full((E_PAD - E,), N, jnp.int32)
    src_rows = jnp.concatenate([src, pad]).reshape(E_PAD // L, L)
    dst_rows = jnp.concatenate([dst, pad]).reshape(E_PAD // L, L)

    y = _typed_transform(x, node_types, W_src)
    a0, a1 = _sc_pass(src_rows, dst_rows, y)
    h1 = _combine(a0, a1, b_src)
    b0, b1 = _sc_pass(src_rows, dst_rows, h1)
    return _final(b0, b1, W_hid, b_hid, W_out, b_out, W_log, b_log)


# speedup vs baseline: 59.1637x; 1.2137x over previous
import functools

import jax
import jax.numpy as jnp
from jax import lax
from jax.experimental import pallas as pl
from jax.experimental.pallas import tpu as pltpu
from jax.experimental.pallas import tpu_sc as plsc

N = 100000
E = 3200000
D = 7
H = 16
OUT = 32
T = 7

NC = 2
NS = 16
NW = NC * NS
L = 128
K = 4
ROWS_PER_W = 800
CHUNKS = ROWS_PER_W // K
E_PAD = NW * ROWS_PER_W * L
BLK = 2048
N_PAD = 100096
GRID = -(-N_PAD // BLK)
TILE_ROWS = N_PAD // NS
ZROWS = TILE_ROWS // 8


def _typed_transform_body(x_ref, nt_ref, w_ref, o_ref):
    i = pl.program_id(0)
    xb = x_ref[...]
    tb = nt_ref[...]
    acc = jnp.zeros((BLK, H), jnp.float32)
    for t in range(T):
        yt = jnp.dot(xb, w_ref[t], preferred_element_type=jnp.float32)
        acc = acc + jnp.where(tb == t, yt, 0.0)
    row = i * BLK + lax.broadcasted_iota(jnp.int32, (BLK, 1), 0)
    o_ref[...] = jnp.where(row < N, acc, 0.0)


def _typed_transform(x, node_types, W_src):
    return pl.pallas_call(
        _typed_transform_body,
        grid=(GRID,),
        in_specs=[
            pl.BlockSpec((BLK, D), lambda i: (i, 0)),
            pl.BlockSpec((BLK, 1), lambda i: (i, 0)),
            pl.BlockSpec((T, D, H), lambda i: (0, 0, 0)),
        ],
        out_specs=pl.BlockSpec((BLK, H), lambda i: (i, 0)),
        out_shape=jax.ShapeDtypeStruct((N_PAD, H), jnp.float32),
    )(x, node_types.reshape(N, 1), W_src)


def _combine_body(p0_ref, p1_ref, b_ref, o_ref):
    o_ref[...] = jax.nn.leaky_relu(p0_ref[...] + p1_ref[...] + b_ref[...])


def _combine(p0, p1, b_src):
    return pl.pallas_call(
        _combine_body,
        grid=(GRID,),
        in_specs=[
            pl.BlockSpec((BLK, H), lambda i: (i, 0)),
            pl.BlockSpec((BLK, H), lambda i: (i, 0)),
            pl.BlockSpec((1, H), lambda i: (0, 0)),
        ],
        out_specs=pl.BlockSpec((BLK, H), lambda i: (i, 0)),
        out_shape=jax.ShapeDtypeStruct((N_PAD, H), jnp.float32),
    )(p0, p1, b_src.reshape(1, H))


def _final_body(p0_ref, p1_ref, wh_ref, bh_ref, wo_ref, bo_ref, wl_ref, bl_ref,
                o_ref, acc_ref):
    i = pl.program_id(0)

    @pl.when(i == 0)
    def _():
        acc_ref[...] = jnp.zeros_like(acc_ref)

    z = p0_ref[...] + p1_ref[...]
    hh = jax.nn.leaky_relu(
        jnp.dot(z, wh_ref[...], preferred_element_type=jnp.float32) + bh_ref[...])
    row = i * BLK + lax.broadcasted_iota(jnp.int32, (BLK, 1), 0)
    hh = jnp.where(row < N, hh, 0.0)
    acc_ref[...] += jnp.sum(hh, axis=0, keepdims=True)

    @pl.when(i == pl.num_programs(0) - 1)
    def _():
        m = acc_ref[...] / N
        g = jax.nn.sigmoid(
            jnp.dot(m, wo_ref[...], preferred_element_type=jnp.float32) + bo_ref[...])
        o_ref[...] = jax.nn.sigmoid(
            jnp.dot(g, wl_ref[...], preferred_element_type=jnp.float32) + bl_ref[...])


def _final(p0, p1, W_hid, b_hid, W_out, b_out, W_log, b_log):
    return pl.pallas_call(
        _final_body,
        grid=(GRID,),
        in_specs=[
            pl.BlockSpec((BLK, H), lambda i: (i, 0)),
            pl.BlockSpec((BLK, H), lambda i: (i, 0)),
            pl.BlockSpec((H, H), lambda i: (0, 0)),
            pl.BlockSpec((1, H), lambda i: (0, 0)),
            pl.BlockSpec((H, OUT), lambda i: (0, 0)),
            pl.BlockSpec((1, OUT), lambda i: (0, 0)),
            pl.BlockSpec((OUT, 1), lambda i: (0, 0)),
            pl.BlockSpec((1, 1), lambda i: (0, 0)),
        ],
        out_specs=pl.BlockSpec((1, 1), lambda i: (0, 0)),
        out_shape=jax.ShapeDtypeStruct((1, 1), jnp.float32),
        scratch_shapes=[pltpu.VMEM((1, H), jnp.float32)],
    )(p0, p1, W_hid, b_hid.reshape(1, H), W_out, b_out.reshape(1, OUT),
      W_log, b_log.reshape(1, 1))


def _sc_pass_kernel(src_hbm, dst_hbm, tab_hbm, out_hbm,
                    sidx, didx, rows, zbuf, acc,
                    isem0, isem1, gsem0, gsem1):
    c = lax.axis_index("c")
    s = lax.axis_index("s")
    wid = s * NC + c

    def zfill(i, carry):
        zbuf[i, :] = jnp.zeros((H,), jnp.float32)
        return carry
    lax.fori_loop(0, ZROWS, zfill, 0)
    tile_base = s * TILE_ROWS
    for q in range(TILE_ROWS // ZROWS):
        pltpu.sync_copy(zbuf, acc.at[pl.ds(tile_base + q * ZROWS, ZROWS), :])
    plsc.subcore_barrier()

    base_row = wid * ROWS_PER_W
    isems = (isem0, isem1)
    gsems = (gsem0, gsem1)

    def idx_copies(chunk_id, b, isem):
        r0 = base_row + chunk_id * K
        return (
            pltpu.make_async_copy(src_hbm.at[pl.ds(r0, K), :], sidx.at[b], isem),
            pltpu.make_async_copy(dst_hbm.at[pl.ds(r0, K), :], didx.at[b], isem),
        )

    def gather_copies(b, gsem):
        return [pltpu.make_async_copy(tab_hbm.at[sidx.at[b, j]],
                                      rows.at[b, j], gsem)
                for j in range(K)]

    def scatter_chunk(b):
        for j in range(K):
            pltpu.sync_copy(rows.at[b, j], acc.at[didx.at[b, j]], add=True)

    for cp in idx_copies(0, 0, isem0):
        cp.start()

    def pair(go, carry):
        for b in range(2):
            cid = 2 * go + b
            for cp in idx_copies(cid, b, isems[b]):
                cp.wait()
            for cp in gather_copies(b, gsems[b]):
                cp.start()
            if b == 0:
                @pl.when(go > 0)
                def _():
                    for cp in gather_copies(1, gsems[1]):
                        cp.wait()
                    scatter_chunk(1)
            else:
                for cp in gather_copies(0, gsems[0]):
                    cp.wait()
                scatter_chunk(0)
            if b == 0:
                for cp in idx_copies(cid + 1, 1, isems[1]):
                    cp.start()
            else:
                @pl.when(cid + 1 < CHUNKS)
                def _():
                    for cp in idx_copies(cid + 1, 0, isems[0]):
                        cp.start()
        return carry
    lax.fori_loop(0, CHUNKS // 2, pair, 0)
    for cp in gather_copies(1, gsems[1]):
        cp.wait()
    scatter_chunk(1)
    plsc.subcore_barrier()

    out_off = c * N_PAD + tile_base
    pltpu.sync_copy(acc.at[pl.ds(tile_base, TILE_ROWS), :],
                    out_hbm.at[pl.ds(out_off, TILE_ROWS), :])


def _sc_pass(src_rows, dst_rows, table):
    mesh = plsc.VectorSubcoreMesh(core_axis_name="c", subcore_axis_name="s")
    k = functools.partial(
        pl.kernel,
        out_type=jax.ShapeDtypeStruct((NC * N_PAD, H), jnp.float32),
        mesh=mesh,
        scratch_types=[
            pltpu.VMEM((2, K, L), jnp.int32),
            pltpu.VMEM((2, K, L), jnp.int32),
            pltpu.VMEM((2, K, L, H), jnp.float32),
            pltpu.VMEM((ZROWS, H), jnp.float32),
            pltpu.VMEM_SHARED((N_PAD, H), jnp.float32),
            pltpu.SemaphoreType.DMA,
            pltpu.SemaphoreType.DMA,
            pltpu.SemaphoreType.DMA,
            pltpu.SemaphoreType.DMA,
        ],
        compiler_params=pltpu.CompilerParams(use_tc_tiling_on_sc=False),
    )(_sc_pass_kernel)
    parts = k(src_rows, dst_rows, table)
    return parts[:N_PAD], parts[N_PAD:]


def kernel(x, edge_index, node_types, W_src, b_src, W_hid, b_hid,
           W_out, b_out, W_log, b_log):
    src = edge_index[0]
    dst = edge_index[1]
    pad = jnp.full((E_PAD - E,), N, jnp.int32)
    src_rows = jnp.concatenate([src, pad]).reshape(E_PAD // L, L)
    dst_rows = jnp.concatenate([dst, pad]).reshape(E_PAD // L, L)

    y = _typed_transform(x, node_types, W_src)
    a0, a1 = _sc_pass(src_rows, dst_rows, y)
    h1 = _combine(a0, a1, b_src)
    b0, b1 = _sc_pass(src_rows, dst_rows, h1)
    return _final(b0, b1, W_hid, b_hid, W_out, b_out, W_log, b_log)


# speedup vs baseline: 63.9302x; 1.0806x over previous
import functools

import jax
import jax.numpy as jnp
from jax import lax
from jax.experimental import pallas as pl
from jax.experimental.pallas import tpu as pltpu
from jax.experimental.pallas import tpu_sc as plsc

N = 100000
E = 3200000
D = 7
H = 16
OUT = 32
T = 7

NC = 2
NS = 16
NW = NC * NS
L = 128
K = 4
ROWS_W0 = 1096
ROWS_W1 = 504
TOTAL_ROWS = (ROWS_W0 + ROWS_W1) * NS
E_PAD = TOTAL_ROWS * L
BLK = 2048
N_PAD = 100096
GRID = -(-N_PAD // BLK)
TILE_ROWS = N_PAD // NS
ZROWS = TILE_ROWS // 8


def _typed_transform_body(x_ref, nt_ref, w_ref, o_ref):
    i = pl.program_id(0)
    xb = x_ref[...]
    tb = nt_ref[...]
    acc = jnp.zeros((BLK, H), jnp.float32)
    for t in range(T):
        yt = jnp.dot(xb, w_ref[t], preferred_element_type=jnp.float32)
        acc = acc + jnp.where(tb == t, yt, 0.0)
    row = i * BLK + lax.broadcasted_iota(jnp.int32, (BLK, 1), 0)
    o_ref[...] = jnp.where(row < N, acc, 0.0)


def _typed_transform(x, node_types, W_src):
    return pl.pallas_call(
        _typed_transform_body,
        grid=(GRID,),
        in_specs=[
            pl.BlockSpec((BLK, D), lambda i: (i, 0)),
            pl.BlockSpec((BLK, 1), lambda i: (i, 0)),
            pl.BlockSpec((T, D, H), lambda i: (0, 0, 0)),
        ],
        out_specs=pl.BlockSpec((BLK, H), lambda i: (i, 0)),
        out_shape=jax.ShapeDtypeStruct((N_PAD, H), jnp.float32),
    )(x, node_types.reshape(N, 1), W_src)


def _combine_body(p0_ref, p1_ref, b_ref, o_ref):
    o_ref[...] = jax.nn.leaky_relu(p0_ref[...] + p1_ref[...] + b_ref[...])


def _combine(p0, p1, b_src):
    return pl.pallas_call(
        _combine_body,
        grid=(GRID,),
        in_specs=[
            pl.BlockSpec((BLK, H), lambda i: (i, 0)),
            pl.BlockSpec((BLK, H), lambda i: (i, 0)),
            pl.BlockSpec((1, H), lambda i: (0, 0)),
        ],
        out_specs=pl.BlockSpec((BLK, H), lambda i: (i, 0)),
        out_shape=jax.ShapeDtypeStruct((N_PAD, H), jnp.float32),
    )(p0, p1, b_src.reshape(1, H))


def _final_body(p0_ref, p1_ref, wh_ref, bh_ref, wo_ref, bo_ref, wl_ref, bl_ref,
                o_ref, acc_ref):
    i = pl.program_id(0)

    @pl.when(i == 0)
    def _():
        acc_ref[...] = jnp.zeros_like(acc_ref)

    z = p0_ref[...] + p1_ref[...]
    hh = jax.nn.leaky_relu(
        jnp.dot(z, wh_ref[...], preferred_element_type=jnp.float32) + bh_ref[...])
    row = i * BLK + lax.broadcasted_iota(jnp.int32, (BLK, 1), 0)
    hh = jnp.where(row < N, hh, 0.0)
    acc_ref[...] += jnp.sum(hh, axis=0, keepdims=True)

    @pl.when(i == pl.num_programs(0) - 1)
    def _():
        m = acc_ref[...] / N
        g = jax.nn.sigmoid(
            jnp.dot(m, wo_ref[...], preferred_element_type=jnp.float32) + bo_ref[...])
        o_ref[...] = jax.nn.sigmoid(
            jnp.dot(g, wl_ref[...], preferred_element_type=jnp.float32) + bl_ref[...])


def _final(p0, p1, W_hid, b_hid, W_out, b_out, W_log, b_log):
    return pl.pallas_call(
        _final_body,
        grid=(GRID,),
        in_specs=[
            pl.BlockSpec((BLK, H), lambda i: (i, 0)),
            pl.BlockSpec((BLK, H), lambda i: (i, 0)),
            pl.BlockSpec((H, H), lambda i: (0, 0)),
            pl.BlockSpec((1, H), lambda i: (0, 0)),
            pl.BlockSpec((H, OUT), lambda i: (0, 0)),
            pl.BlockSpec((1, OUT), lambda i: (0, 0)),
            pl.BlockSpec((OUT, 1), lambda i: (0, 0)),
            pl.BlockSpec((1, 1), lambda i: (0, 0)),
        ],
        out_specs=pl.BlockSpec((1, 1), lambda i: (0, 0)),
        out_shape=jax.ShapeDtypeStruct((1, 1), jnp.float32),
        scratch_shapes=[pltpu.VMEM((1, H), jnp.float32)],
    )(p0, p1, W_hid, b_hid.reshape(1, H), W_out, b_out.reshape(1, OUT),
      W_log, b_log.reshape(1, 1))


def _sc_pass_kernel(src_hbm, dst_hbm, tab_hbm, out_hbm,
                    sidx, didx, rows, zbuf, acc,
                    isem0, isem1, gsem0, gsem1):
    c = lax.axis_index("c")
    s = lax.axis_index("s")
    wid = s * NC + c

    def zfill(i, carry):
        zbuf[i, :] = jnp.zeros((H,), jnp.float32)
        return carry
    lax.fori_loop(0, ZROWS, zfill, 0)
    tile_base = s * TILE_ROWS
    for q in range(TILE_ROWS // ZROWS):
        pltpu.sync_copy(zbuf, acc.at[pl.ds(tile_base + q * ZROWS, ZROWS), :])
    plsc.subcore_barrier()

    base_row = jnp.where(c == 0, s * ROWS_W0, NS * ROWS_W0 + s * ROWS_W1)
    nchunks = jnp.where(c == 0, ROWS_W0 // K, ROWS_W1 // K)
    isems = (isem0, isem1)
    gsems = (gsem0, gsem1)

    def idx_copies(chunk_id, b, isem):
        r0 = base_row + chunk_id * K
        return (
            pltpu.make_async_copy(src_hbm.at[pl.ds(r0, K), :], sidx.at[b], isem),
            pltpu.make_async_copy(dst_hbm.at[pl.ds(r0, K), :], didx.at[b], isem),
        )

    def gather_copies(b, gsem):
        return [pltpu.make_async_copy(tab_hbm.at[sidx.at[b, j]],
                                      rows.at[b, j], gsem)
                for j in range(K)]

    def scatter_chunk(b):
        for j in range(K):
            pltpu.sync_copy(rows.at[b, j], acc.at[didx.at[b, j]], add=True)

    for cp in idx_copies(0, 0, isem0):
        cp.start()

    def pair(go, carry):
        for b in range(2):
            cid = 2 * go + b
            for cp in idx_copies(cid, b, isems[b]):
                cp.wait()
            for cp in gather_copies(b, gsems[b]):
                cp.start()
            if b == 0:
                @pl.when(go > 0)
                def _():
                    for cp in gather_copies(1, gsems[1]):
                        cp.wait()
                    scatter_chunk(1)
            else:
                for cp in gather_copies(0, gsems[0]):
                    cp.wait()
                scatter_chunk(0)
            if b == 0:
                for cp in idx_copies(cid + 1, 1, isems[1]):
                    cp.start()
            else:
                @pl.when(cid + 1 < nchunks)
                def _():
                    for cp in idx_copies(cid + 1, 0, isems[0]):
                        cp.start()
        return carry
    lax.fori_loop(0, nchunks // 2, pair, 0)
    for cp in gather_copies(1, gsems[1]):
        cp.wait()
    scatter_chunk(1)
    plsc.subcore_barrier()

    out_off = c * N_PAD + tile_base
    pltpu.sync_copy(acc.at[pl.ds(tile_base, TILE_ROWS), :],
                    out_hbm.at[pl.ds(out_off, TILE_ROWS), :])


def _sc_pass(src_rows, dst_rows, table):
    mesh = plsc.VectorSubcoreMesh(core_axis_name="c", subcore_axis_name="s")
    k = functools.partial(
        pl.kernel,
        out_type=jax.ShapeDtypeStruct((NC * N_PAD, H), jnp.float32),
        mesh=mesh,
        scratch_types=[
            pltpu.VMEM((2, K, L), jnp.int32),
            pltpu.VMEM((2, K, L), jnp.int32),
            pltpu.VMEM((2, K, L, H), jnp.float32),
            pltpu.VMEM((ZROWS, H), jnp.float32),
            pltpu.VMEM_SHARED((N_PAD, H), jnp.float32),
            pltpu.SemaphoreType.DMA,
            pltpu.SemaphoreType.DMA,
            pltpu.SemaphoreType.DMA,
            pltpu.SemaphoreType.DMA,
        ],
        compiler_params=pltpu.CompilerParams(use_tc_tiling_on_sc=False),
    )(_sc_pass_kernel)
    parts = k(src_rows, dst_rows, table)
    return parts[:N_PAD], parts[N_PAD:]


def kernel(x, edge_index, node_types, W_src, b_src, W_hid, b_hid,
           W_out, b_out, W_log, b_log):
    src = edge_index[0]
    dst = edge_index[1]
    pad = jnp.full((E_PAD - E,), N, jnp.int32)
    src_rows = jnp.concatenate([src, pad]).reshape(E_PAD // L, L)
    dst_rows = jnp.concatenate([dst, pad]).reshape(E_PAD // L, L)

    y = _typed_transform(x, node_types, W_src)
    a0, a1 = _sc_pass(src_rows, dst_rows, y)
    h1 = _combine(a0, a1, b_src)
    b0, b1 = _sc_pass(src_rows, dst_rows, h1)
    return _final(b0, b1, W_hid, b_hid, W_out, b_out, W_log, b_log)


# speedup vs baseline: 66.0660x; 1.0334x over previous
import functools

import jax
import jax.numpy as jnp
from jax import lax
from jax.experimental import pallas as pl
from jax.experimental.pallas import tpu as pltpu
from jax.experimental.pallas import tpu_sc as plsc

N = 100000
E = 3200000
D = 7
H = 16
OUT = 32
T = 7

NC = 2
NS = 16
NW = NC * NS
L = 128
K = 4
ROWS_W0 = 1360
ROWS_W1 = 240
TOTAL_ROWS = (ROWS_W0 + ROWS_W1) * NS
E_PAD = TOTAL_ROWS * L
BLK = 2048
N_PAD = 100096
GRID = -(-N_PAD // BLK)
TILE_ROWS = N_PAD // NS
ZROWS = TILE_ROWS // 8


def _typed_transform_body(x_ref, nt_ref, w_ref, o_ref):
    i = pl.program_id(0)
    xb = x_ref[...]
    tb = nt_ref[...]
    acc = jnp.zeros((BLK, H), jnp.float32)
    for t in range(T):
        yt = jnp.dot(xb, w_ref[t], preferred_element_type=jnp.float32)
        acc = acc + jnp.where(tb == t, yt, 0.0)
    row = i * BLK + lax.broadcasted_iota(jnp.int32, (BLK, 1), 0)
    o_ref[...] = jnp.where(row < N, acc, 0.0)


def _typed_transform(x, node_types, W_src):
    return pl.pallas_call(
        _typed_transform_body,
        grid=(GRID,),
        in_specs=[
            pl.BlockSpec((BLK, D), lambda i: (i, 0)),
            pl.BlockSpec((BLK, 1), lambda i: (i, 0)),
            pl.BlockSpec((T, D, H), lambda i: (0, 0, 0)),
        ],
        out_specs=pl.BlockSpec((BLK, H), lambda i: (i, 0)),
        out_shape=jax.ShapeDtypeStruct((N_PAD, H), jnp.float32),
    )(x, node_types.reshape(N, 1), W_src)


def _combine_body(p0_ref, p1_ref, b_ref, o_ref):
    o_ref[...] = jax.nn.leaky_relu(p0_ref[...] + p1_ref[...] + b_ref[...])


def _combine(p0, p1, b_src):
    return pl.pallas_call(
        _combine_body,
        grid=(GRID,),
        in_specs=[
            pl.BlockSpec((BLK, H), lambda i: (i, 0)),
            pl.BlockSpec((BLK, H), lambda i: (i, 0)),
            pl.BlockSpec((1, H), lambda i: (0, 0)),
        ],
        out_specs=pl.BlockSpec((BLK, H), lambda i: (i, 0)),
        out_shape=jax.ShapeDtypeStruct((N_PAD, H), jnp.float32),
    )(p0, p1, b_src.reshape(1, H))


def _final_body(p0_ref, p1_ref, wh_ref, bh_ref, wo_ref, bo_ref, wl_ref, bl_ref,
                o_ref, acc_ref):
    i = pl.program_id(0)

    @pl.when(i == 0)
    def _():
        acc_ref[...] = jnp.zeros_like(acc_ref)

    z = p0_ref[...] + p1_ref[...]
    hh = jax.nn.leaky_relu(
        jnp.dot(z, wh_ref[...], preferred_element_type=jnp.float32) + bh_ref[...])
    row = i * BLK + lax.broadcasted_iota(jnp.int32, (BLK, 1), 0)
    hh = jnp.where(row < N, hh, 0.0)
    acc_ref[...] += jnp.sum(hh, axis=0, keepdims=True)

    @pl.when(i == pl.num_programs(0) - 1)
    def _():
        m = acc_ref[...] / N
        g = jax.nn.sigmoid(
            jnp.dot(m, wo_ref[...], preferred_element_type=jnp.float32) + bo_ref[...])
        o_ref[...] = jax.nn.sigmoid(
            jnp.dot(g, wl_ref[...], preferred_element_type=jnp.float32) + bl_ref[...])


def _final(p0, p1, W_hid, b_hid, W_out, b_out, W_log, b_log):
    return pl.pallas_call(
        _final_body,
        grid=(GRID,),
        in_specs=[
            pl.BlockSpec((BLK, H), lambda i: (i, 0)),
            pl.BlockSpec((BLK, H), lambda i: (i, 0)),
            pl.BlockSpec((H, H), lambda i: (0, 0)),
            pl.BlockSpec((1, H), lambda i: (0, 0)),
            pl.BlockSpec((H, OUT), lambda i: (0, 0)),
            pl.BlockSpec((1, OUT), lambda i: (0, 0)),
            pl.BlockSpec((OUT, 1), lambda i: (0, 0)),
            pl.BlockSpec((1, 1), lambda i: (0, 0)),
        ],
        out_specs=pl.BlockSpec((1, 1), lambda i: (0, 0)),
        out_shape=jax.ShapeDtypeStruct((1, 1), jnp.float32),
        scratch_shapes=[pltpu.VMEM((1, H), jnp.float32)],
    )(p0, p1, W_hid, b_hid.reshape(1, H), W_out, b_out.reshape(1, OUT),
      W_log, b_log.reshape(1, 1))


def _sc_pass_kernel(src_hbm, dst_hbm, tab_hbm, out_hbm,
                    sidx, didx, rows, zbuf, acc,
                    isem0, isem1, gsem0, gsem1):
    c = lax.axis_index("c")
    s = lax.axis_index("s")
    wid = s * NC + c

    def zfill(i, carry):
        zbuf[i, :] = jnp.zeros((H,), jnp.float32)
        return carry
    lax.fori_loop(0, ZROWS, zfill, 0)
    tile_base = s * TILE_ROWS
    for q in range(TILE_ROWS // ZROWS):
        pltpu.sync_copy(zbuf, acc.at[pl.ds(tile_base + q * ZROWS, ZROWS), :])
    plsc.subcore_barrier()

    base_row = jnp.where(c == 0, s * ROWS_W0, NS * ROWS_W0 + s * ROWS_W1)
    nchunks = jnp.where(c == 0, ROWS_W0 // K, ROWS_W1 // K)
    isems = (isem0, isem1)
    gsems = (gsem0, gsem1)

    def idx_copies(chunk_id, b, isem):
        r0 = base_row + chunk_id * K
        return (
            pltpu.make_async_copy(src_hbm.at[pl.ds(r0, K), :], sidx.at[b], isem),
            pltpu.make_async_copy(dst_hbm.at[pl.ds(r0, K), :], didx.at[b], isem),
        )

    def gather_copies(b, gsem):
        return [pltpu.make_async_copy(tab_hbm.at[sidx.at[b, j]],
                                      rows.at[b, j], gsem)
                for j in range(K)]

    def scatter_chunk(b):
        for j in range(K):
            pltpu.sync_copy(rows.at[b, j], acc.at[didx.at[b, j]], add=True)

    for cp in idx_copies(0, 0, isem0):
        cp.start()

    def pair(go, carry):
        for b in range(2):
            cid = 2 * go + b
            for cp in idx_copies(cid, b, isems[b]):
                cp.wait()
            for cp in gather_copies(b, gsems[b]):
                cp.start()
            if b == 0:
                @pl.when(go > 0)
                def _():
                    for cp in gather_copies(1, gsems[1]):
                        cp.wait()
                    scatter_chunk(1)
            else:
                for cp in gather_copies(0, gsems[0]):
                    cp.wait()
                scatter_chunk(0)
            if b == 0:
                for cp in idx_copies(cid + 1, 1, isems[1]):
                    cp.start()
            else:
                @pl.when(cid + 1 < nchunks)
                def _():
                    for cp in idx_copies(cid + 1, 0, isems[0]):
                        cp.start()
        return carry
    lax.fori_loop(0, nchunks // 2, pair, 0)
    for cp in gather_copies(1, gsems[1]):
        cp.wait()
    scatter_chunk(1)
    plsc.subcore_barrier()

    out_off = c * N_PAD + tile_base
    pltpu.sync_copy(acc.at[pl.ds(tile_base, TILE_ROWS), :],
                    out_hbm.at[pl.ds(out_off, TILE_ROWS), :])


def _sc_pass(src_rows, dst_rows, table):
    mesh = plsc.VectorSubcoreMesh(core_axis_name="c", subcore_axis_name="s")
    k = functools.partial(
        pl.kernel,
        out_type=jax.ShapeDtypeStruct((NC * N_PAD, H), jnp.float32),
        mesh=mesh,
        scratch_types=[
            pltpu.VMEM((2, K, L), jnp.int32),
            pltpu.VMEM((2, K, L), jnp.int32),
            pltpu.VMEM((2, K, L, H), jnp.float32),
            pltpu.VMEM((ZROWS, H), jnp.float32),
            pltpu.VMEM_SHARED((N_PAD, H), jnp.float32),
            pltpu.SemaphoreType.DMA,
            pltpu.SemaphoreType.DMA,
            pltpu.SemaphoreType.DMA,
            pltpu.SemaphoreType.DMA,
        ],
        compiler_params=pltpu.CompilerParams(use_tc_tiling_on_sc=False),
    )(_sc_pass_kernel)
    parts = k(src_rows, dst_rows, table)
    return parts[:N_PAD], parts[N_PAD:]


def kernel(x, edge_index, node_types, W_src, b_src, W_hid, b_hid,
           W_out, b_out, W_log, b_log):
    src = edge_index[0]
    dst = edge_index[1]
    pad = jnp.full((E_PAD - E,), N, jnp.int32)
    src_rows = jnp.concatenate([src, pad]).reshape(E_PAD // L, L)
    dst_rows = jnp.concatenate([dst, pad]).reshape(E_PAD // L, L)

    y = _typed_transform(x, node_types, W_src)
    a0, a1 = _sc_pass(src_rows, dst_rows, y)
    h1 = _combine(a0, a1, b_src)
    b0, b1 = _sc_pass(src_rows, dst_rows, h1)
    return _final(b0, b1, W_hid, b_hid, W_out, b_out, W_log, b_log)


# speedup vs baseline: 80.3718x; 1.2165x over previous
import functools

import jax
import jax.numpy as jnp
from jax import lax
from jax.experimental import pallas as pl
from jax.experimental.pallas import tpu as pltpu
from jax.experimental.pallas import tpu_sc as plsc

N = 100000
E = 3200000
D = 7
H = 16
OUT = 32
T = 7

NC = 2
NS = 16
L = 125
K = 4
ROWS_W0 = 1360
ROWS_W1 = 240
TOTAL_ROWS = (ROWS_W0 + ROWS_W1) * NS
BLK = 2048
N_PAD = 100096
GRID = -(-N_PAD // BLK)
NP8 = N_PAD // 8
TILE_ROWS = N_PAD // NS
ZROWS = TILE_ROWS // 8


def _typed_transform_body(x_ref, nt_ref, wcat_ref, o_ref):
    i = pl.program_id(0)
    xb = x_ref[...]
    tb = nt_ref[...]
    y_all = jnp.dot(xb, wcat_ref[...], preferred_element_type=jnp.float32)
    acc = jnp.zeros((BLK, H), jnp.float32)
    for t in range(T):
        acc = acc + jnp.where(tb == t, y_all[:, 16 * t:16 * t + 16], 0.0)
    row = i * BLK + lax.broadcasted_iota(jnp.int32, (BLK, 1), 0)
    o_ref[...] = jnp.where(row < N, acc, 0.0)


def _typed_transform(x, node_types, W_cat):
    return pl.pallas_call(
        _typed_transform_body,
        grid=(GRID,),
        in_specs=[
            pl.BlockSpec((BLK, D), lambda i: (i, 0)),
            pl.BlockSpec((BLK, 1), lambda i: (i, 0)),
            pl.BlockSpec((D, T * H), lambda i: (0, 0)),
        ],
        out_specs=pl.BlockSpec((BLK, H), lambda i: (i, 0)),
        out_shape=jax.ShapeDtypeStruct((N_PAD, H), jnp.float32),
    )(x, node_types.reshape(N, 1), W_cat)


def _combine_body(pp_ref, b_ref, o_ref):
    pp = pp_ref[...]
    o_ref[...] = jax.nn.leaky_relu(pp[:NP8] + pp[NP8:] + b_ref[...])


def _combine(parts, b128):
    h128 = pl.pallas_call(
        _combine_body,
        grid=(1,),
        in_specs=[
            pl.BlockSpec((2 * NP8, 128), lambda i: (0, 0)),
            pl.BlockSpec((1, 128), lambda i: (0, 0)),
        ],
        out_specs=pl.BlockSpec((NP8, 128), lambda i: (0, 0)),
        out_shape=jax.ShapeDtypeStruct((NP8, 128), jnp.float32),
    )(parts.reshape(2 * NP8, 128), b128)
    return h128.reshape(N_PAD, H)


def _final_body(pp_ref, wh_ref, bh_ref, wo_ref, bo_ref, wl_ref, bl_ref, o_ref):
    pp = pp_ref[...]
    z = pp[:NP8] + pp[NP8:]
    u = jax.nn.leaky_relu(
        jnp.dot(z, wh_ref[...], preferred_element_type=jnp.float32) + bh_ref[...])
    row = lax.broadcasted_iota(jnp.int32, (NP8, 1), 0)
    u = jnp.where(row < N // 8, u, 0.0)
    s = jnp.sum(u, axis=0, keepdims=True)
    m = sum(s[:, 16 * g:16 * g + 16] for g in range(8)) / N
    g = jax.nn.sigmoid(
        jnp.dot(m, wo_ref[...], preferred_element_type=jnp.float32) + bo_ref[...])
    o_ref[...] = jax.nn.sigmoid(
        jnp.dot(g, wl_ref[...], preferred_element_type=jnp.float32) + bl_ref[...])


def _final(parts, W128, b128h, W_out, b_out, W_log, b_log):
    return pl.pallas_call(
        _final_body,
        grid=(1,),
        in_specs=[
            pl.BlockSpec((2 * NP8, 128), lambda i: (0, 0)),
            pl.BlockSpec((128, 128), lambda i: (0, 0)),
            pl.BlockSpec((1, 128), lambda i: (0, 0)),
            pl.BlockSpec((H, OUT), lambda i: (0, 0)),
            pl.BlockSpec((1, OUT), lambda i: (0, 0)),
            pl.BlockSpec((OUT, 1), lambda i: (0, 0)),
            pl.BlockSpec((1, 1), lambda i: (0, 0)),
        ],
        out_specs=pl.BlockSpec((1, 1), lambda i: (0, 0)),
        out_shape=jax.ShapeDtypeStruct((1, 1), jnp.float32),
    )(parts.reshape(2 * NP8, 128), W128, b128h, W_out, b_out.reshape(1, OUT),
      W_log, b_log.reshape(1, 1))


def _sc_pass_kernel(src_hbm, dst_hbm, tab_hbm, out_hbm,
                    sidx, didx, rows, zbuf, acc,
                    isem0, isem1, gsem0, gsem1):
    c = lax.axis_index("c")
    s = lax.axis_index("s")

    def zfill(i, carry):
        zbuf[i, :] = jnp.zeros((H,), jnp.float32)
        return carry
    lax.fori_loop(0, ZROWS, zfill, 0)
    tile_base = s * TILE_ROWS
    for q in range(TILE_ROWS // ZROWS):
        pltpu.sync_copy(zbuf, acc.at[pl.ds(tile_base + q * ZROWS, ZROWS), :])
    plsc.subcore_barrier()

    base_row = jnp.where(c == 0, s * ROWS_W0, NS * ROWS_W0 + s * ROWS_W1)
    nchunks = jnp.where(c == 0, ROWS_W0 // K, ROWS_W1 // K)
    isems = (isem0, isem1)
    gsems = (gsem0, gsem1)

    def idx_copies(chunk_id, b, isem):
        r0 = base_row + chunk_id * K
        return (
            pltpu.make_async_copy(src_hbm.at[pl.ds(r0, K), :], sidx.at[b], isem),
            pltpu.make_async_copy(dst_hbm.at[pl.ds(r0, K), :], didx.at[b], isem),
        )

    def gather_copies(b, gsem):
        return [pltpu.make_async_copy(tab_hbm.at[sidx.at[b, j]],
                                      rows.at[b, j], gsem)
                for j in range(K)]

    def scatter_chunk(b):
        for j in range(K):
            pltpu.sync_copy(rows.at[b, j], acc.at[didx.at[b, j]], add=True)

    for cp in idx_copies(0, 0, isem0):
        cp.start()

    def pair(go, carry):
        for b in range(2):
            cid = 2 * go + b
            for cp in idx_copies(cid, b, isems[b]):
                cp.wait()
            for cp in gather_copies(b, gsems[b]):
                cp.start()
            if b == 0:
                @pl.when(go > 0)
                def _():
                    for cp in gather_copies(1, gsems[1]):
                        cp.wait()
                    scatter_chunk(1)
            else:
                for cp in gather_copies(0, gsems[0]):
                    cp.wait()
                scatter_chunk(0)
            if b == 0:
                for cp in idx_copies(cid + 1, 1, isems[1]):
                    cp.start()
            else:
                @pl.when(cid + 1 < nchunks)
                def _():
                    for cp in idx_copies(cid + 1, 0, isems[0]):
                        cp.start()
        return carry
    lax.fori_loop(0, nchunks // 2, pair, 0)
    for cp in gather_copies(1, gsems[1]):
        cp.wait()
    scatter_chunk(1)
    plsc.subcore_barrier()

    out_off = c * N_PAD + tile_base
    pltpu.sync_copy(acc.at[pl.ds(tile_base, TILE_ROWS), :],
                    out_hbm.at[pl.ds(out_off, TILE_ROWS), :])


def _sc_pass(src_rows, dst_rows, table):
    mesh = plsc.VectorSubcoreMesh(core_axis_name="c", subcore_axis_name="s")
    k = functools.partial(
        pl.kernel,
        out_type=jax.ShapeDtypeStruct((NC * N_PAD, H), jnp.float32),
        mesh=mesh,
        scratch_types=[
            pltpu.VMEM((2, K, L), jnp.int32),
            pltpu.VMEM((2, K, L), jnp.int32),
            pltpu.VMEM((2, K, L, H), jnp.float32),
            pltpu.VMEM((ZROWS, H), jnp.float32),
            pltpu.VMEM_SHARED((N_PAD, H), jnp.float32),
            pltpu.SemaphoreType.DMA,
            pltpu.SemaphoreType.DMA,
            pltpu.SemaphoreType.DMA,
            pltpu.SemaphoreType.DMA,
        ],
        compiler_params=pltpu.CompilerParams(use_tc_tiling_on_sc=False),
    )(_sc_pass_kernel)
    return k(src_rows, dst_rows, table)


def kernel(x, edge_index, node_types, W_src, b_src, W_hid, b_hid,
           W_out, b_out, W_log, b_log):
    src_rows = edge_index[0].reshape(TOTAL_ROWS, L)
    dst_rows = edge_index[1].reshape(TOTAL_ROWS, L)
    W_cat = jnp.transpose(W_src, (1, 0, 2)).reshape(D, T * H)
    b128s = jnp.tile(b_src, 8).reshape(1, 128)
    W128 = jnp.kron(jnp.eye(8, dtype=jnp.float32), W_hid)
    b128h = jnp.tile(b_hid, 8).reshape(1, 128)

    y = _typed_transform(x, node_types, W_cat)
    parts1 = _sc_pass(src_rows, dst_rows, y)
    h1 = _combine(parts1, b128s)
    parts2 = _sc_pass(src_rows, dst_rows, h1)
    return _final(parts2, W128, b128h, W_out, b_out, W_log, b_log)


# speedup vs baseline: 135.1603x; 1.6817x over previous
import functools

import jax
import jax.numpy as jnp
from jax import lax
from jax.experimental import pallas as pl
from jax.experimental.pallas import tpu as pltpu
from jax.experimental.pallas import tpu_sc as plsc

N = 100000
E = 3200000
D = 7
H = 16
OUT = 32
T = 7

NC = 2
NS = 16
L = 128
K = 4
TOTAL_ROWS = E // L
U0 = 107
U0_EXTRA = 5
U1 = 88
C0_UNITS = U0 * NS + U0_EXTRA
assert C0_UNITS + U1 * NS == TOTAL_ROWS // (2 * K)
BLK = 2048
N_PAD = 100096
GRID = -(-N_PAD // BLK)
NP8 = N_PAD // 8
TILE_ROWS = N_PAD // NS
ZROWS = TILE_ROWS // 8


def _typed_transform_body(x_ref, nt_ref, w_ref, o_ref):
    i = pl.program_id(0)
    xb = x_ref[...]
    tb = nt_ref[...]
    acc = jnp.zeros((BLK, H), jnp.float32)
    for t in range(T):
        yt = jnp.dot(xb, w_ref[t], preferred_element_type=jnp.float32)
        acc = acc + jnp.where(tb == t, yt, 0.0)
    row = i * BLK + lax.broadcasted_iota(jnp.int32, (BLK, 1), 0)
    o_ref[...] = jnp.where(row < N, acc, 0.0)


def _typed_transform(x, node_types, W_src):
    return pl.pallas_call(
        _typed_transform_body,
        grid=(GRID,),
        in_specs=[
            pl.BlockSpec((BLK, D), lambda i: (i, 0)),
            pl.BlockSpec((BLK, 1), lambda i: (i, 0)),
            pl.BlockSpec((T, D, H), lambda i: (0, 0, 0)),
        ],
        out_specs=pl.BlockSpec((BLK, H), lambda i: (i, 0)),
        out_shape=jax.ShapeDtypeStruct((N_PAD, H), jnp.float32),
    )(x, node_types.reshape(N, 1), W_src)


def _combine_body(pp_ref, b_ref, o_ref):
    pp = pp_ref[...]
    o_ref[...] = jax.nn.leaky_relu(pp[:NP8] + pp[NP8:] + b_ref[...])


def _combine(parts, b128):
    h128 = pl.pallas_call(
        _combine_body,
        grid=(1,),
        in_specs=[
            pl.BlockSpec((2 * NP8, 128), lambda i: (0, 0)),
            pl.BlockSpec((1, 128), lambda i: (0, 0)),
        ],
        out_specs=pl.BlockSpec((NP8, 128), lambda i: (0, 0)),
        out_shape=jax.ShapeDtypeStruct((NP8, 128), jnp.float32),
    )(parts.reshape(2 * NP8, 128), b128)
    return h128.reshape(N_PAD, H)


def _final_body(pp_ref, wh_ref, bh_ref, wo_ref, bo_ref, wl_ref, bl_ref, o_ref):
    pp = pp_ref[...]
    z = pp[:NP8] + pp[NP8:]
    u = jax.nn.leaky_relu(
        jnp.dot(z, wh_ref[...], preferred_element_type=jnp.float32) + bh_ref[...])
    row = lax.broadcasted_iota(jnp.int32, (NP8, 1), 0)
    u = jnp.where(row < N // 8, u, 0.0)
    s = jnp.sum(u, axis=0, keepdims=True)
    m = sum(s[:, 16 * g:16 * g + 16] for g in range(8)) / N
    g = jax.nn.sigmoid(
        jnp.dot(m, wo_ref[...], preferred_element_type=jnp.float32) + bo_ref[...])
    o_ref[...] = jax.nn.sigmoid(
        jnp.dot(g, wl_ref[...], preferred_element_type=jnp.float32) + bl_ref[...])


def _final(parts, W128, b128h, W_out, b_out, W_log, b_log):
    return pl.pallas_call(
        _final_body,
        grid=(1,),
        in_specs=[
            pl.BlockSpec((2 * NP8, 128), lambda i: (0, 0)),
            pl.BlockSpec((128, 128), lambda i: (0, 0)),
            pl.BlockSpec((1, 128), lambda i: (0, 0)),
            pl.BlockSpec((H, OUT), lambda i: (0, 0)),
            pl.BlockSpec((1, OUT), lambda i: (0, 0)),
            pl.BlockSpec((OUT, 1), lambda i: (0, 0)),
            pl.BlockSpec((1, 1), lambda i: (0, 0)),
        ],
        out_specs=pl.BlockSpec((1, 1), lambda i: (0, 0)),
        out_shape=jax.ShapeDtypeStruct((1, 1), jnp.float32),
    )(parts.reshape(2 * NP8, 128), W128, b128h, W_out, b_out.reshape(1, OUT),
      W_log, b_log.reshape(1, 1))


def _sc_pass_kernel(ei_hbm, tab_hbm, out_hbm,
                    sidx, didx, rows, zbuf, acc,
                    isem0, isem1, gsem0, gsem1):
    c = lax.axis_index("c")
    s = lax.axis_index("s")

    def zfill(i, carry):
        zbuf[i, :] = jnp.zeros((H,), jnp.float32)
        return carry
    lax.fori_loop(0, ZROWS, zfill, 0)
    tile_base = s * TILE_ROWS
    for q in range(TILE_ROWS // ZROWS):
        pltpu.sync_copy(zbuf, acc.at[pl.ds(tile_base + q * ZROWS, ZROWS), :])
    plsc.subcore_barrier()

    base_units = jnp.where(c == 0, s * U0 + jnp.minimum(s, U0_EXTRA),
                           C0_UNITS + s * U1)
    units = jnp.where(c == 0, U0 + (s < U0_EXTRA).astype(jnp.int32), U1)
    base_row = base_units * (2 * K)
    nchunks = units * 2
    isems = (isem0, isem1)
    gsems = (gsem0, gsem1)

    def idx_copies(chunk_id, b, isem):
        r0 = base_row + chunk_id * K
        return (
            pltpu.make_async_copy(ei_hbm.at[0, pl.ds(r0, K), :], sidx.at[b], isem),
            pltpu.make_async_copy(ei_hbm.at[1, pl.ds(r0, K), :], didx.at[b], isem),
        )

    def gather_copies(b, gsem):
        return [pltpu.make_async_copy(tab_hbm.at[sidx.at[b, j]],
                                      rows.at[b, j], gsem)
                for j in range(K)]

    def scatter_chunk(b):
        for j in range(K):
            pltpu.sync_copy(rows.at[b, j], acc.at[didx.at[b, j]], add=True)

    for cp in idx_copies(0, 0, isem0):
        cp.start()

    def pair(go, carry):
        for b in range(2):
            cid = 2 * go + b
            for cp in idx_copies(cid, b, isems[b]):
                cp.wait()
            for cp in gather_copies(b, gsems[b]):
                cp.start()
            if b == 0:
                @pl.when(go > 0)
                def _():
                    for cp in gather_copies(1, gsems[1]):
                        cp.wait()
                    scatter_chunk(1)
            else:
                for cp in gather_copies(0, gsems[0]):
                    cp.wait()
                scatter_chunk(0)
            if b == 0:
                for cp in idx_copies(cid + 1, 1, isems[1]):
                    cp.start()
            else:
                @pl.when(cid + 1 < nchunks)
                def _():
                    for cp in idx_copies(cid + 1, 0, isems[0]):
                        cp.start()
        return carry
    lax.fori_loop(0, nchunks // 2, pair, 0)
    for cp in gather_copies(1, gsems[1]):
        cp.wait()
    scatter_chunk(1)
    plsc.subcore_barrier()

    out_off = c * N_PAD + tile_base
    pltpu.sync_copy(acc.at[pl.ds(tile_base, TILE_ROWS), :],
                    out_hbm.at[pl.ds(out_off, TILE_ROWS), :])


def _sc_pass(ei3, table):
    mesh = plsc.VectorSubcoreMesh(core_axis_name="c", subcore_axis_name="s")
    k = functools.partial(
        pl.kernel,
        out_type=jax.ShapeDtypeStruct((NC * N_PAD, H), jnp.float32),
        mesh=mesh,
        scratch_types=[
            pltpu.VMEM((2, K, L), jnp.int32),
            pltpu.VMEM((2, K, L), jnp.int32),
            pltpu.VMEM((2, K, L, H), jnp.float32),
            pltpu.VMEM((ZROWS, H), jnp.float32),
            pltpu.VMEM_SHARED((N_PAD, H), jnp.float32),
            pltpu.SemaphoreType.DMA,
            pltpu.SemaphoreType.DMA,
            pltpu.SemaphoreType.DMA,
            pltpu.SemaphoreType.DMA,
        ],
        compiler_params=pltpu.CompilerParams(use_tc_tiling_on_sc=False),
    )(_sc_pass_kernel)
    return k(ei3, table)


def kernel(x, edge_index, node_types, W_src, b_src, W_hid, b_hid,
           W_out, b_out, W_log, b_log):
    ei3 = edge_index.reshape(2, TOTAL_ROWS, L)
    b128s = jnp.tile(b_src, 8).reshape(1, 128)
    W128 = jnp.kron(jnp.eye(8, dtype=jnp.float32), W_hid)
    b128h = jnp.tile(b_hid, 8).reshape(1, 128)

    y = _typed_transform(x, node_types, W_src)
    parts1 = _sc_pass(ei3, y)
    h1 = _combine(parts1, b128s)
    parts2 = _sc_pass(ei3, h1)
    return _final(parts2, W128, b128h, W_out, b_out, W_log, b_log)


# speedup vs baseline: 143.7829x; 1.0638x over previous
import functools

import jax
import jax.numpy as jnp
from jax import lax
from jax.experimental import pallas as pl
from jax.experimental.pallas import tpu as pltpu
from jax.experimental.pallas import tpu_sc as plsc

N = 100000
E = 3200000
D = 7
H = 16
OUT = 32
T = 7

NC = 2
NS = 16
L = 128
K = 4
TOTAL_ROWS = E // L
U0 = 97
U0_EXTRA = 10
U1 = 97
U1_EXTRA = 11
C0_UNITS = U0 * NS + U0_EXTRA
assert C0_UNITS + U1 * NS + U1_EXTRA == TOTAL_ROWS // (2 * K)
BLK = 2048
N_PAD = 100096
GRID = -(-N_PAD // BLK)
NP8 = N_PAD // 8
TILE_ROWS = N_PAD // NS
ZROWS = TILE_ROWS // 8


def _typed_transform_body(x_ref, nt_ref, w_ref, o_ref):
    i = pl.program_id(0)
    xb = x_ref[...]
    tb = nt_ref[...]
    acc = jnp.zeros((BLK, H), jnp.float32)
    for t in range(T):
        yt = jnp.dot(xb, w_ref[t], preferred_element_type=jnp.float32)
        acc = acc + jnp.where(tb == t, yt, 0.0)
    row = i * BLK + lax.broadcasted_iota(jnp.int32, (BLK, 1), 0)
    o_ref[...] = jnp.where(row < N, acc, 0.0)


def _typed_transform(x, node_types, W_src):
    return pl.pallas_call(
        _typed_transform_body,
        grid=(GRID,),
        in_specs=[
            pl.BlockSpec((BLK, D), lambda i: (i, 0)),
            pl.BlockSpec((BLK, 1), lambda i: (i, 0)),
            pl.BlockSpec((T, D, H), lambda i: (0, 0, 0)),
        ],
        out_specs=pl.BlockSpec((BLK, H), lambda i: (i, 0)),
        out_shape=jax.ShapeDtypeStruct((N_PAD, H), jnp.float32),
    )(x, node_types.reshape(N, 1), W_src)


def _combine_body(pp_ref, b_ref, o_ref):
    pp = pp_ref[...]
    o_ref[...] = jax.nn.leaky_relu(pp[:NP8] + pp[NP8:] + b_ref[...])


def _combine(parts, b128):
    h128 = pl.pallas_call(
        _combine_body,
        grid=(1,),
        in_specs=[
            pl.BlockSpec((2 * NP8, 128), lambda i: (0, 0)),
            pl.BlockSpec((1, 128), lambda i: (0, 0)),
        ],
        out_specs=pl.BlockSpec((NP8, 128), lambda i: (0, 0)),
        out_shape=jax.ShapeDtypeStruct((NP8, 128), jnp.float32),
    )(parts.reshape(2 * NP8, 128), b128)
    return h128.reshape(N_PAD, H)


def _final_body(pp_ref, wh_ref, bh_ref, wo_ref, bo_ref, wl_ref, bl_ref, o_ref):
    pp = pp_ref[...]
    z = pp[:NP8] + pp[NP8:]
    u = jax.nn.leaky_relu(
        jnp.dot(z, wh_ref[...], preferred_element_type=jnp.float32) + bh_ref[...])
    row = lax.broadcasted_iota(jnp.int32, (NP8, 1), 0)
    u = jnp.where(row < N // 8, u, 0.0)
    s = jnp.sum(u, axis=0, keepdims=True)
    m = sum(s[:, 16 * g:16 * g + 16] for g in range(8)) / N
    g = jax.nn.sigmoid(
        jnp.dot(m, wo_ref[...], preferred_element_type=jnp.float32) + bo_ref[...])
    o_ref[...] = jax.nn.sigmoid(
        jnp.dot(g, wl_ref[...], preferred_element_type=jnp.float32) + bl_ref[...])


def _final(parts, W128, b128h, W_out, b_out, W_log, b_log):
    return pl.pallas_call(
        _final_body,
        grid=(1,),
        in_specs=[
            pl.BlockSpec((2 * NP8, 128), lambda i: (0, 0)),
            pl.BlockSpec((128, 128), lambda i: (0, 0)),
            pl.BlockSpec((1, 128), lambda i: (0, 0)),
            pl.BlockSpec((H, OUT), lambda i: (0, 0)),
            pl.BlockSpec((1, OUT), lambda i: (0, 0)),
            pl.BlockSpec((OUT, 1), lambda i: (0, 0)),
            pl.BlockSpec((1, 1), lambda i: (0, 0)),
        ],
        out_specs=pl.BlockSpec((1, 1), lambda i: (0, 0)),
        out_shape=jax.ShapeDtypeStruct((1, 1), jnp.float32),
    )(parts.reshape(2 * NP8, 128), W128, b128h, W_out, b_out.reshape(1, OUT),
      W_log, b_log.reshape(1, 1))


def _sc_pass_kernel(ei_hbm, tab_hbm, out_hbm,
                    sidx, didx, rows, zbuf, acc,
                    isem0, isem1, gsem0, gsem1):
    c = lax.axis_index("c")
    s = lax.axis_index("s")

    def zfill(i, carry):
        zbuf[i, :] = jnp.zeros((H,), jnp.float32)
        return carry
    lax.fori_loop(0, ZROWS, zfill, 0)
    tile_base = s * TILE_ROWS
    for q in range(TILE_ROWS // ZROWS):
        pltpu.sync_copy(zbuf, acc.at[pl.ds(tile_base + q * ZROWS, ZROWS), :])
    plsc.subcore_barrier()

    base_units = jnp.where(c == 0, s * U0 + jnp.minimum(s, U0_EXTRA),
                           C0_UNITS + s * U1 + jnp.minimum(s, U1_EXTRA))
    units = jnp.where(c == 0, U0 + (s < U0_EXTRA).astype(jnp.int32),
                      U1 + (s < U1_EXTRA).astype(jnp.int32))
    base_row = base_units * (2 * K)
    nchunks = units * 2
    isems = (isem0, isem1)
    gsems = (gsem0, gsem1)

    def idx_copies(chunk_id, b, isem):
        r0 = base_row + chunk_id * K
        return (
            pltpu.make_async_copy(ei_hbm.at[0, pl.ds(r0, K), :], sidx.at[b], isem),
            pltpu.make_async_copy(ei_hbm.at[1, pl.ds(r0, K), :], didx.at[b], isem),
        )

    def gather_copies(b, gsem):
        return [pltpu.make_async_copy(tab_hbm.at[sidx.at[b, j]],
                                      rows.at[b, j], gsem)
                for j in range(K)]

    def scatter_chunk(b):
        for j in range(K):
            pltpu.sync_copy(rows.at[b, j], acc.at[didx.at[b, j]], add=True)

    for cp in idx_copies(0, 0, isem0):
        cp.start()

    def pair(go, carry):
        for b in range(2):
            cid = 2 * go + b
            for cp in idx_copies(cid, b, isems[b]):
                cp.wait()
            for cp in gather_copies(b, gsems[b]):
                cp.start()
            if b == 0:
                @pl.when(go > 0)
                def _():
                    for cp in gather_copies(1, gsems[1]):
                        cp.wait()
                    scatter_chunk(1)
            else:
                for cp in gather_copies(0, gsems[0]):
                    cp.wait()
                scatter_chunk(0)
            if b == 0:
                for cp in idx_copies(cid + 1, 1, isems[1]):
                    cp.start()
            else:
                @pl.when(cid + 1 < nchunks)
                def _():
                    for cp in idx_copies(cid + 1, 0, isems[0]):
                        cp.start()
        return carry
    lax.fori_loop(0, nchunks // 2, pair, 0)
    for cp in gather_copies(1, gsems[1]):
        cp.wait()
    scatter_chunk(1)
    plsc.subcore_barrier()

    out_off = c * N_PAD + tile_base
    pltpu.sync_copy(acc.at[pl.ds(tile_base, TILE_ROWS), :],
                    out_hbm.at[pl.ds(out_off, TILE_ROWS), :])


def _sc_pass(ei3, table):
    mesh = plsc.VectorSubcoreMesh(core_axis_name="c", subcore_axis_name="s")
    k = functools.partial(
        pl.kernel,
        out_type=jax.ShapeDtypeStruct((NC * N_PAD, H), jnp.float32),
        mesh=mesh,
        scratch_types=[
            pltpu.VMEM((2, K, L), jnp.int32),
            pltpu.VMEM((2, K, L), jnp.int32),
            pltpu.VMEM((2, K, L, H), jnp.float32),
            pltpu.VMEM((ZROWS, H), jnp.float32),
            pltpu.VMEM_SHARED((N_PAD, H), jnp.float32),
            pltpu.SemaphoreType.DMA,
            pltpu.SemaphoreType.DMA,
            pltpu.SemaphoreType.DMA,
            pltpu.SemaphoreType.DMA,
        ],
        compiler_params=pltpu.CompilerParams(use_tc_tiling_on_sc=False),
    )(_sc_pass_kernel)
    return k(ei3, table)


def kernel(x, edge_index, node_types, W_src, b_src, W_hid, b_hid,
           W_out, b_out, W_log, b_log):
    ei3 = edge_index.reshape(2, TOTAL_ROWS, L)
    b128s = jnp.tile(b_src, 8).reshape(1, 128)
    W128 = jnp.kron(jnp.eye(8, dtype=jnp.float32), W_hid)
    b128h = jnp.tile(b_hid, 8).reshape(1, 128)

    y = _typed_transform(x, node_types, W_src)
    parts1 = _sc_pass(ei3, y)
    h1 = _combine(parts1, b128s)
    parts2 = _sc_pass(ei3, h1)
    return _final(parts2, W128, b128h, W_out, b_out, W_log, b_log)
